# Initial kernel scaffold; baseline (speedup 1.0000x reference)
import jax
import jax.numpy as jnp
from jax.experimental import pallas as pl


def kernel(x):
    raise NotImplementedError("write your pallas kernel here")


# speedup vs baseline: 6.7199x; 6.7199x over previous
import jax
import jax.numpy as jnp
from jax import lax
from jax.experimental import pallas as pl

_K = 16
_DIL = 2
_NEEDED = _K * _DIL - 1
_ROWS = 256


def _knn_body(xfull_ref, xrows_ref, out_ref):
    xf = xfull_ref[0]
    xr = xrows_ref[0]
    nf = jnp.sqrt(jnp.sum(xf * xf, axis=-1, keepdims=True))
    xfn = xf / jnp.maximum(nf, 1e-12)
    nr = jnp.sqrt(jnp.sum(xr * xr, axis=-1, keepdims=True))
    xrn = xr / jnp.maximum(nr, 1e-12)
    sqf = jnp.sum(xfn * xfn, axis=-1)
    sqr = jnp.sum(xrn * xrn, axis=-1)
    inner = -2.0 * lax.dot_general(
        xrn, xfn, (((1,), (1,)), ((), ())),
        preferred_element_type=jnp.float32)
    dist = (sqr[:, None] + inner) + sqf[None, :]
    R, N = dist.shape
    iota = lax.broadcasted_iota(jnp.int32, (R, N), 1)
    work = dist
    cols = []
    inf = jnp.float32(jnp.inf)
    for t in range(_NEEDED):
        m = jnp.min(work, axis=1, keepdims=True)
        idx = jnp.min(jnp.where(work == m, iota, N), axis=1,
                      keepdims=True)
        if t % _DIL == 0:
            cols.append(idx)
        if t + 1 < _NEEDED:
            work = jnp.where(iota == idx, inf, work)
    out_ref[0] = jnp.concatenate(cols, axis=1)


def kernel(x):
    B, C, N, _ = x.shape
    xt = jnp.squeeze(jnp.transpose(x, (0, 2, 1, 3)), -1)
    grid = (B, N // _ROWS)
    nn = pl.pallas_call(
        _knn_body,
        grid=grid,
        in_specs=[
            pl.BlockSpec((1, N, C), lambda b, r: (b, 0, 0)),
            pl.BlockSpec((1, _ROWS, C), lambda b, r: (b, r, 0)),
        ],
        out_specs=pl.BlockSpec((1, _ROWS, _K), lambda b, r: (b, r, 0)),
        out_shape=jax.ShapeDtypeStruct((B, N, _K), jnp.int32),
    )(xt, xt)
    center = jnp.broadcast_to(
        jnp.arange(N, dtype=jnp.int32)[None, :, None], (B, N, _K))
    return jnp.stack([nn, center], axis=0)


# speedup vs baseline: 8.1243x; 1.2090x over previous
import functools

import jax
import jax.numpy as jnp
from jax import lax
from jax.experimental import pallas as pl
from jax.experimental.pallas import tpu as pltpu
from jax.experimental.pallas import tpu_sc as plsc

_K = 16
_B = 4
_C = 32
_N = 4096
_ROWS = 256
_L = 16
_NW = 32

_SPLIT = 12800
_TC_TILES = _SPLIT // _ROWS
_ALL_TILES = (_B * _N) // _ROWS
_SC_ROWS = _B * _N - _SPLIT
_RPW = _SC_ROWS // _NW


def _normalize(xfull_ref, xrows_ref):
    xf = xfull_ref[0]
    xr = xrows_ref[0]
    nf = jnp.sqrt(jnp.sum(xf * xf, axis=-1, keepdims=True))
    xfn = xf / jnp.maximum(nf, 1e-12)
    nr = jnp.sqrt(jnp.sum(xr * xr, axis=-1, keepdims=True))
    xrn = xr / jnp.maximum(nr, 1e-12)
    return xfn, xrn


def _dist_tile(xfn, xrn):
    sqf = jnp.sum(xfn * xfn, axis=-1)
    sqr = jnp.sum(xrn * xrn, axis=-1)
    inner = -2.0 * lax.dot_general(
        xrn, xfn, (((1,), (1,)), ((), ())),
        preferred_element_type=jnp.float32)
    return (sqr[:, None] + inner) + sqf[None, :]


def _dist_body(xfull_ref, xrows_ref, out_ref):
    xfn, xrn = _normalize(xfull_ref, xrows_ref)
    out_ref[...] = _dist_tile(xfn, xrn)


def _fused_body(xfull_ref, xrows_ref, out_ref):
    xfn, xrn = _normalize(xfull_ref, xrows_ref)
    dist = _dist_tile(xfn, xrn)
    R, N = dist.shape
    iota = lax.broadcasted_iota(jnp.int32, (R, N), 1)
    work = dist
    cols = []
    inf = jnp.float32(jnp.inf)
    for t in range(2 * _K - 1):
        m = jnp.min(work, axis=1, keepdims=True)
        idx = jnp.min(jnp.where(work == m, iota, N), axis=1,
                      keepdims=True)
        if t % 2 == 0:
            cols.append(idx)
        if t + 1 < 2 * _K - 1:
            work = jnp.where(iota == idx, inf, work)
    out_ref[...] = jnp.concatenate(cols, axis=1)


def _lexlt(ak, ai, bk, bi):
    return (ak < bk) | ((ak == bk) & (ai < bi))


def _split(ak, ai, bk, bi):
    m = _lexlt(ak, ai, bk, bi)
    lok = jnp.where(m, ak, bk)
    loi = jnp.where(m, ai, bi)
    hik = jnp.where(m, bk, ak)
    hii = jnp.where(m, bi, ai)
    return lok, loi, hik, hii


def _rev(x):
    return lax.rev(x, (0,))


def _topk_row(buf, p, r, outb, stage):
    lane = lax.iota(jnp.int32, _L)
    inf = jnp.float32(jnp.inf)

    def chunk(t, carry):
        c0k, c0i, c1k, c1i, th = carry
        v = buf[p, pl.ds(t * _L, _L)]
        cmin = jnp.min(v)

        def merge(op):
            c0k, c0i, c1k, c1i, _ = op
            idx = lane + t * _L
            sv, si = plsc.sort_key_val(v, idx)
            lo1k, lo1i, hi1k, hi1i = _split(c1k, c1i, _rev(sv), _rev(si))
            lo1k, lo1i = plsc.sort_key_val(lo1k, lo1i)
            hi1k, hi1i = plsc.sort_key_val(hi1k, hi1i)
            nc0k, nc0i, hi2k, hi2i = _split(c0k, c0i, _rev(lo1k), _rev(lo1i))
            nc0k, nc0i = plsc.sort_key_val(nc0k, nc0i)
            hi2k, hi2i = plsc.sort_key_val(hi2k, hi2i)
            nc1k, nc1i, _uk, _ui = _split(hi2k, hi2i, _rev(hi1k), _rev(hi1i))
            nc1k, nc1i = plsc.sort_key_val(nc1k, nc1i)
            return nc0k, nc0i, nc1k, nc1i, jnp.max(nc1k)

        return lax.cond(cmin <= th, merge, lambda op: op, carry)

    init = (jnp.full((_L,), inf, jnp.float32), jnp.zeros((_L,), jnp.int32),
            jnp.full((_L,), inf, jnp.float32), jnp.zeros((_L,), jnp.int32),
            inf)
    c0k, c0i, c1k, c1i, _ = lax.fori_loop(0, _N // _L, chunk, init)
    stage[0] = c0i
    stage[1] = c1i
    ev = plsc.load_gather(stage, [lane // 8, (lane * 2) % _L])
    outb[r] = ev


_mesh = plsc.VectorSubcoreMesh(core_axis_name="c", subcore_axis_name="s")


@functools.partial(
    pl.kernel,
    mesh=_mesh,
    compiler_params=pltpu.CompilerParams(needs_layout_passes=False),
    out_type=jax.ShapeDtypeStruct((_SC_ROWS, _L), jnp.int32),
    scratch_types=[
        pltpu.VMEM((2, _N), jnp.float32),
        pltpu.VMEM((_RPW, _L), jnp.int32),
        pltpu.VMEM((2, _L), jnp.int32),
        pltpu.SemaphoreType.DMA,
        pltpu.SemaphoreType.DMA,
    ],
)
def _sc_topk(dist_hbm, out_hbm, buf, outb, stage, sem0, sem1):
    cid = lax.axis_index("c")
    sid = lax.axis_index("s")
    wid = sid * 2 + cid
    base = wid * _RPW
    pltpu.async_copy(dist_hbm.at[base], buf.at[0], sem0)

    def row_loop(g, carry):
        for p in range(2):
            r = g * 2 + p
            sem_cur = sem0 if p == 0 else sem1
            sem_nxt = sem1 if p == 0 else sem0

            @pl.when(r + 1 < _RPW)
            def _():
                pltpu.async_copy(dist_hbm.at[base + r + 1], buf.at[1 - p],
                                 sem_nxt)

            pltpu.make_async_copy(dist_hbm.at[base + r], buf.at[p],
                                  sem_cur).wait()
            _topk_row(buf, p, r, outb, stage)
        return carry

    lax.fori_loop(0, _RPW // 2, row_loop, 0)
    pltpu.sync_copy(outb, out_hbm.at[pl.ds(base, _RPW)])


def kernel(x):
    B, C, N, _ = x.shape
    xt = jnp.squeeze(jnp.transpose(x, (0, 2, 1, 3)), -1)
    tpr = N // _ROWS
    dist_sc = pl.pallas_call(
        _dist_body,
        grid=(_ALL_TILES - _TC_TILES,),
        in_specs=[
            pl.BlockSpec((1, N, _C), lambda i: ((i + _TC_TILES) // tpr, 0, 0)),
            pl.BlockSpec((1, _ROWS, _C),
                         lambda i: ((i + _TC_TILES) // tpr,
                                    (i + _TC_TILES) % tpr, 0)),
        ],
        out_specs=pl.BlockSpec((_ROWS, N), lambda i: (i, 0)),
        out_shape=jax.ShapeDtypeStruct((_SC_ROWS, N), jnp.float32),
    )(xt, xt)
    nn_sc = _sc_topk(dist_sc)
    nn_tc = pl.pallas_call(
        _fused_body,
        grid=(_TC_TILES,),
        in_specs=[
            pl.BlockSpec((1, N, _C), lambda i: (i // tpr, 0, 0)),
            pl.BlockSpec((1, _ROWS, _C), lambda i: (i // tpr, i % tpr, 0)),
        ],
        out_specs=pl.BlockSpec((_ROWS, _K), lambda i: (i, 0)),
        out_shape=jax.ShapeDtypeStruct((_SPLIT, _K), jnp.int32),
    )(xt, xt)
    nn = jnp.concatenate([nn_tc, nn_sc], axis=0).reshape(B, N, _K)
    center = jnp.broadcast_to(
        jnp.arange(N, dtype=jnp.int32)[None, :, None], (B, N, _K))
    return jnp.stack([nn, center], axis=0)


# speedup vs baseline: 10.0048x; 1.2315x over previous
import functools

import jax
import jax.numpy as jnp
from jax import lax
from jax.experimental import pallas as pl
from jax.experimental.pallas import tpu as pltpu
from jax.experimental.pallas import tpu_sc as plsc

_K = 16
_B = 4
_C = 32
_N = 4096
_ROWS = 256
_L = 16
_NW = 32

_SPLIT = 10240
_TC_TILES = _SPLIT // _ROWS
_ALL_TILES = (_B * _N) // _ROWS
_SC_ROWS = _B * _N - _SPLIT
_RPW = _SC_ROWS // _NW


def _normalize(xfull_ref, xrows_ref):
    xf = xfull_ref[0]
    xr = xrows_ref[0]
    nf = jnp.sqrt(jnp.sum(xf * xf, axis=-1, keepdims=True))
    xfn = xf / jnp.maximum(nf, 1e-12)
    nr = jnp.sqrt(jnp.sum(xr * xr, axis=-1, keepdims=True))
    xrn = xr / jnp.maximum(nr, 1e-12)
    return xfn, xrn


def _dist_tile(xfn, xrn):
    sqf = jnp.sum(xfn * xfn, axis=-1)
    sqr = jnp.sum(xrn * xrn, axis=-1)
    inner = -2.0 * lax.dot_general(
        xrn, xfn, (((1,), (1,)), ((), ())),
        preferred_element_type=jnp.float32)
    return (sqr[:, None] + inner) + sqf[None, :]


def _dist_body(xfull_ref, xrows_ref, out_ref):
    xfn, xrn = _normalize(xfull_ref, xrows_ref)
    out_ref[...] = _dist_tile(xfn, xrn)


def _fused_body(xfull_ref, xrows_ref, out_ref):
    xfn, xrn = _normalize(xfull_ref, xrows_ref)
    dist = _dist_tile(xfn, xrn)
    R, N = dist.shape
    iota = lax.broadcasted_iota(jnp.int32, (R, N), 1)
    work = dist
    cols = []
    inf = jnp.float32(jnp.inf)
    for t in range(2 * _K - 1):
        m = jnp.min(work, axis=1, keepdims=True)
        idx = jnp.min(jnp.where(work == m, iota, N), axis=1,
                      keepdims=True)
        if t % 2 == 0:
            cols.append(idx)
        if t + 1 < 2 * _K - 1:
            work = jnp.where(iota == idx, inf, work)
    out_ref[...] = jnp.concatenate(cols, axis=1)


def _lexlt(ak, ai, bk, bi):
    return (ak < bk) | ((ak == bk) & (ai < bi))


def _split(ak, ai, bk, bi):
    m = _lexlt(ak, ai, bk, bi)
    lok = jnp.where(m, ak, bk)
    loi = jnp.where(m, ai, bi)
    hik = jnp.where(m, bk, ak)
    hii = jnp.where(m, bi, ai)
    return lok, loi, hik, hii


def _rev(x):
    return lax.rev(x, (0,))


def _merge_chunk(c0k, c0i, c1k, c1i, vk, vi):
    sv, si = plsc.sort_key_val(vk, vi)
    lo1k, lo1i, hi1k, hi1i = _split(c1k, c1i, _rev(sv), _rev(si))
    lo1k, lo1i = plsc.sort_key_val(lo1k, lo1i)
    hi1k, hi1i = plsc.sort_key_val(hi1k, hi1i)
    nc0k, nc0i, hi2k, hi2i = _split(c0k, c0i, _rev(lo1k), _rev(lo1i))
    nc0k, nc0i = plsc.sort_key_val(nc0k, nc0i)
    hi2k, hi2i = plsc.sort_key_val(hi2k, hi2i)
    nc1k, nc1i, _uk, _ui = _split(hi2k, hi2i, _rev(hi1k), _rev(hi1i))
    nc1k, nc1i = plsc.sort_key_val(nc1k, nc1i)
    return nc0k, nc0i, nc1k, nc1i


_GRP = 4
_CAP = 112
_TRIG = 48


def _topk_row(buf, p, r, outb, stage, abk, abi):
    lane = lax.iota(jnp.int32, _L)
    inf = jnp.float32(jnp.inf)

    def consolidate(c0k, c0i, c1k, c1i, ncnt):
        ncnt_s = jnp.broadcast_to(ncnt, (_L,))
        st = (c0k, c0i, c1k, c1i)

        def fold(st, j):
            bk = abk[pl.ds(j * _L, _L)]
            bi = abi[pl.ds(j * _L, _L)]
            valid = (lane + j * _L) < ncnt_s
            bk = jnp.where(valid, bk, inf)
            bi = jnp.where(valid, bi, 0)
            return _merge_chunk(*st, bk, bi)

        for j in range(_CAP // _L):
            st = lax.cond(j * _L < ncnt, lambda s: fold(s, j),
                          lambda s: s, st)
        return st

    def group(g, carry):
        c0k, c0i, c1k, c1i, th, cnt = carry
        vs = [buf[p, pl.ds((g * _GRP + j) * _L, _L)] for j in range(_GRP)]
        gmin = jnp.min(jnp.minimum(jnp.minimum(vs[0], vs[1]),
                                   jnp.minimum(vs[2], vs[3])))

        def hit(op):
            c0k, c0i, c1k, c1i, th, cnt = op
            th_s = jnp.broadcast_to(th, (_L,))
            off = jnp.broadcast_to(cnt, (_L,))
            for j in range(_GRP):
                m = vs[j] <= th_s
                pos = off + plsc.cumsum(m.astype(jnp.int32)) - 1
                plsc.store_scatter(abk, [pos], vs[j], mask=m)
                plsc.store_scatter(abi, [pos],
                                   lane + (g * _GRP + j) * _L, mask=m)
                off = off + plsc.all_reduce_population_count(m)
            ncnt = off[0]

            def consol(op2):
                c0k, c0i, c1k, c1i, _th, _c = op2
                c0k, c0i, c1k, c1i = consolidate(c0k, c0i, c1k, c1i, ncnt)
                return (c0k, c0i, c1k, c1i, jnp.max(c1k),
                        jnp.int32(0))

            return lax.cond(
                ncnt >= _TRIG, consol,
                lambda op2: (op2[0], op2[1], op2[2], op2[3], op2[4], ncnt),
                (c0k, c0i, c1k, c1i, th, cnt))

        return lax.cond(gmin <= th, hit, lambda op: op, carry)

    init = (jnp.full((_L,), inf, jnp.float32), jnp.zeros((_L,), jnp.int32),
            jnp.full((_L,), inf, jnp.float32), jnp.zeros((_L,), jnp.int32),
            inf, jnp.int32(0))
    c0k, c0i, c1k, c1i, th, cnt = lax.fori_loop(
        0, _N // (_GRP * _L), group, init)
    c0k, c0i, c1k, c1i = lax.cond(
        cnt > 0,
        lambda st: consolidate(st[0], st[1], st[2], st[3], cnt),
        lambda st: st, (c0k, c0i, c1k, c1i))
    stage[0] = c0i
    stage[1] = c1i
    ev = plsc.load_gather(stage, [lane // 8, (lane * 2) % _L])
    outb[r] = ev


_mesh = plsc.VectorSubcoreMesh(core_axis_name="c", subcore_axis_name="s")


@functools.partial(
    pl.kernel,
    mesh=_mesh,
    compiler_params=pltpu.CompilerParams(needs_layout_passes=False),
    out_type=jax.ShapeDtypeStruct((_SC_ROWS, _L), jnp.int32),
    scratch_types=[
        pltpu.VMEM((2, _N), jnp.float32),
        pltpu.VMEM((_RPW, _L), jnp.int32),
        pltpu.VMEM((2, _L), jnp.int32),
        pltpu.VMEM((_CAP,), jnp.float32),
        pltpu.VMEM((_CAP,), jnp.int32),
        pltpu.SemaphoreType.DMA,
        pltpu.SemaphoreType.DMA,
    ],
)
def _sc_topk(dist_hbm, out_hbm, buf, outb, stage, abk, abi, sem0, sem1):
    cid = lax.axis_index("c")
    sid = lax.axis_index("s")
    wid = sid * 2 + cid
    base = wid * _RPW
    pltpu.async_copy(dist_hbm.at[base], buf.at[0], sem0)

    def row_loop(g, carry):
        for p in range(2):
            r = g * 2 + p
            sem_cur = sem0 if p == 0 else sem1
            sem_nxt = sem1 if p == 0 else sem0

            @pl.when(r + 1 < _RPW)
            def _():
                pltpu.async_copy(dist_hbm.at[base + r + 1], buf.at[1 - p],
                                 sem_nxt)

            pltpu.make_async_copy(dist_hbm.at[base + r], buf.at[p],
                                  sem_cur).wait()
            _topk_row(buf, p, r, outb, stage, abk, abi)
        return carry

    lax.fori_loop(0, _RPW // 2, row_loop, 0)
    pltpu.sync_copy(outb, out_hbm.at[pl.ds(base, _RPW)])


def kernel(x):
    B, C, N, _ = x.shape
    xt = jnp.squeeze(jnp.transpose(x, (0, 2, 1, 3)), -1)
    tpr = N // _ROWS
    dist_sc = pl.pallas_call(
        _dist_body,
        grid=(_ALL_TILES - _TC_TILES,),
        in_specs=[
            pl.BlockSpec((1, N, _C), lambda i: ((i + _TC_TILES) // tpr, 0, 0)),
            pl.BlockSpec((1, _ROWS, _C),
                         lambda i: ((i + _TC_TILES) // tpr,
                                    (i + _TC_TILES) % tpr, 0)),
        ],
        out_specs=pl.BlockSpec((_ROWS, N), lambda i: (i, 0)),
        out_shape=jax.ShapeDtypeStruct((_SC_ROWS, N), jnp.float32),
    )(xt, xt)
    nn_sc = _sc_topk(dist_sc)
    nn_tc = pl.pallas_call(
        _fused_body,
        grid=(_TC_TILES,),
        in_specs=[
            pl.BlockSpec((1, N, _C), lambda i: (i // tpr, 0, 0)),
            pl.BlockSpec((1, _ROWS, _C), lambda i: (i // tpr, i % tpr, 0)),
        ],
        out_specs=pl.BlockSpec((_ROWS, _K), lambda i: (i, 0)),
        out_shape=jax.ShapeDtypeStruct((_SPLIT, _K), jnp.int32),
    )(xt, xt)
    nn = jnp.concatenate([nn_tc, nn_sc], axis=0).reshape(B, N, _K)
    center = jnp.broadcast_to(
        jnp.arange(N, dtype=jnp.int32)[None, :, None], (B, N, _K))
    return jnp.stack([nn, center], axis=0)


# speedup vs baseline: 12.4013x; 1.2395x over previous
import functools

import jax
import jax.numpy as jnp
from jax import lax
from jax.experimental import pallas as pl
from jax.experimental.pallas import tpu as pltpu
from jax.experimental.pallas import tpu_sc as plsc

_K = 16
_B = 4
_C = 32
_N = 4096
_ROWS = 256
_L = 16
_NW = 32

_SPLIT = 7936
_TC_TILES = _SPLIT // _ROWS
_ALL_TILES = (_B * _N) // _ROWS
_SC_ROWS = _B * _N - _SPLIT
_RPW = _SC_ROWS // _NW


def _normalize(xfull_ref, xrows_ref):
    xf = xfull_ref[0]
    xr = xrows_ref[0]
    nf = jnp.sqrt(jnp.sum(xf * xf, axis=-1, keepdims=True))
    xfn = xf / jnp.maximum(nf, 1e-12)
    nr = jnp.sqrt(jnp.sum(xr * xr, axis=-1, keepdims=True))
    xrn = xr / jnp.maximum(nr, 1e-12)
    return xfn, xrn


def _dist_tile(xfn, xrn):
    sqf = jnp.sum(xfn * xfn, axis=-1)
    sqr = jnp.sum(xrn * xrn, axis=-1)
    inner = -2.0 * lax.dot_general(
        xrn, xfn, (((1,), (1,)), ((), ())),
        preferred_element_type=jnp.float32)
    return (sqr[:, None] + inner) + sqf[None, :]


def _dist_body(xfull_ref, xrows_ref, out_ref):
    xfn, xrn = _normalize(xfull_ref, xrows_ref)
    out_ref[...] = _dist_tile(xfn, xrn)


def _fused_body(xfull_ref, xrows_ref, out_ref):
    xfn, xrn = _normalize(xfull_ref, xrows_ref)
    dist = _dist_tile(xfn, xrn)
    R, N = dist.shape
    iota = lax.broadcasted_iota(jnp.int32, (R, N), 1)
    work = dist
    cols = []
    inf = jnp.float32(jnp.inf)
    for t in range(2 * _K - 1):
        m = jnp.min(work, axis=1, keepdims=True)
        idx = jnp.min(jnp.where(work == m, iota, N), axis=1,
                      keepdims=True)
        if t % 2 == 0:
            cols.append(idx)
        if t + 1 < 2 * _K - 1:
            work = jnp.where(iota == idx, inf, work)
    out_ref[...] = jnp.concatenate(cols, axis=1)


def _lexlt(ak, ai, bk, bi):
    return (ak < bk) | ((ak == bk) & (ai < bi))


def _split(ak, ai, bk, bi):
    m = _lexlt(ak, ai, bk, bi)
    lok = jnp.where(m, ak, bk)
    loi = jnp.where(m, ai, bi)
    hik = jnp.where(m, bk, ak)
    hii = jnp.where(m, bi, ai)
    return lok, loi, hik, hii


def _rev(x):
    return lax.rev(x, (0,))


def _merge_chunk(c0k, c0i, c1k, c1i, vk, vi):
    sv, si = plsc.sort_key_val(vk, vi)
    lo1k, lo1i, hi1k, hi1i = _split(c1k, c1i, _rev(sv), _rev(si))
    lo1k, lo1i = plsc.sort_key_val(lo1k, lo1i)
    hi1k, hi1i = plsc.sort_key_val(hi1k, hi1i)
    nc0k, nc0i, hi2k, hi2i = _split(c0k, c0i, _rev(lo1k), _rev(lo1i))
    nc0k, nc0i = plsc.sort_key_val(nc0k, nc0i)
    hi2k, hi2i = plsc.sort_key_val(hi2k, hi2i)
    nc1k, nc1i, _uk, _ui = _split(hi2k, hi2i, _rev(hi1k), _rev(hi1i))
    nc1k, nc1i = plsc.sort_key_val(nc1k, nc1i)
    return nc0k, nc0i, nc1k, nc1i


_GRP = 4
_CAP = 112
_TRIG = 48


def _topk_row(buf, p, r, outb, stage, abk, abi):
    lane = lax.iota(jnp.int32, _L)
    inf = jnp.float32(jnp.inf)

    def consolidate(c0k, c0i, c1k, c1i, ncnt):
        ncnt_s = jnp.broadcast_to(ncnt, (_L,))
        st = (c0k, c0i, c1k, c1i)

        def fold(st, j):
            bk = abk[pl.ds(j * _L, _L)]
            bi = abi[pl.ds(j * _L, _L)]
            valid = (lane + j * _L) < ncnt_s
            bk = jnp.where(valid, bk, inf)
            bi = jnp.where(valid, bi, 0)
            return _merge_chunk(*st, bk, bi)

        for j in range(_CAP // _L):
            st = lax.cond(j * _L < ncnt, lambda s: fold(s, j),
                          lambda s: s, st)
        return st

    def group(g, carry):
        c0k, c0i, c1k, c1i, th, cnt = carry
        vs = [buf[p, pl.ds((g * _GRP + j) * _L, _L)] for j in range(_GRP)]
        gmin = jnp.min(jnp.minimum(jnp.minimum(vs[0], vs[1]),
                                   jnp.minimum(vs[2], vs[3])))

        def hit(op):
            c0k, c0i, c1k, c1i, th, cnt = op
            th_s = jnp.broadcast_to(th, (_L,))
            off = jnp.broadcast_to(cnt, (_L,))
            for j in range(_GRP):
                m = vs[j] <= th_s
                pos = off + plsc.cumsum(m.astype(jnp.int32)) - 1
                plsc.store_scatter(abk, [pos], vs[j], mask=m)
                plsc.store_scatter(abi, [pos],
                                   lane + (g * _GRP + j) * _L, mask=m)
                off = off + plsc.all_reduce_population_count(m)
            ncnt = off[0]

            def consol(op2):
                c0k, c0i, c1k, c1i, _th, _c = op2
                c0k, c0i, c1k, c1i = consolidate(c0k, c0i, c1k, c1i, ncnt)
                return (c0k, c0i, c1k, c1i, jnp.max(c1k),
                        jnp.int32(0))

            return lax.cond(
                ncnt >= _TRIG, consol,
                lambda op2: (op2[0], op2[1], op2[2], op2[3], op2[4], ncnt),
                (c0k, c0i, c1k, c1i, th, cnt))

        return lax.cond(gmin <= th, hit, lambda op: op, carry)

    init = (jnp.full((_L,), inf, jnp.float32), jnp.zeros((_L,), jnp.int32),
            jnp.full((_L,), inf, jnp.float32), jnp.zeros((_L,), jnp.int32),
            inf, jnp.int32(0))
    c0k, c0i, c1k, c1i, th, cnt = lax.fori_loop(
        0, _N // (_GRP * _L), group, init)
    c0k, c0i, c1k, c1i = lax.cond(
        cnt > 0,
        lambda st: consolidate(st[0], st[1], st[2], st[3], cnt),
        lambda st: st, (c0k, c0i, c1k, c1i))
    stage[0] = c0i
    stage[1] = c1i
    ev = plsc.load_gather(stage, [lane // 8, (lane * 2) % _L])
    outb[r] = ev


_mesh = plsc.VectorSubcoreMesh(core_axis_name="c", subcore_axis_name="s")


@functools.partial(
    pl.kernel,
    mesh=_mesh,
    compiler_params=pltpu.CompilerParams(needs_layout_passes=False),
    out_type=jax.ShapeDtypeStruct((_SC_ROWS, _L), jnp.int32),
    scratch_types=[
        pltpu.VMEM((2, _N), jnp.float32),
        pltpu.VMEM((_RPW, _L), jnp.int32),
        pltpu.VMEM((2, _L), jnp.int32),
        pltpu.VMEM((_CAP,), jnp.float32),
        pltpu.VMEM((_CAP,), jnp.int32),
        pltpu.SemaphoreType.DMA,
        pltpu.SemaphoreType.DMA,
    ],
)
def _sc_topk(dist_hbm, out_hbm, buf, outb, stage, abk, abi, sem0, sem1):
    cid = lax.axis_index("c")
    sid = lax.axis_index("s")
    wid = sid * 2 + cid
    base = wid * _RPW
    pltpu.async_copy(dist_hbm.at[base], buf.at[0], sem0)

    def row_loop(g, carry):
        for p in range(2):
            r = g * 2 + p
            sem_cur = sem0 if p == 0 else sem1
            sem_nxt = sem1 if p == 0 else sem0

            @pl.when(r + 1 < _RPW)
            def _():
                pltpu.async_copy(dist_hbm.at[base + r + 1], buf.at[1 - p],
                                 sem_nxt)

            pltpu.make_async_copy(dist_hbm.at[base + r], buf.at[p],
                                  sem_cur).wait()
            _topk_row(buf, p, r, outb, stage, abk, abi)
        return carry

    lax.fori_loop(0, _RPW // 2, row_loop, 0)
    pltpu.sync_copy(outb, out_hbm.at[pl.ds(base, _RPW)])


def kernel(x):
    B, C, N, _ = x.shape
    xt = jnp.squeeze(jnp.transpose(x, (0, 2, 1, 3)), -1)
    tpr = N // _ROWS
    dist_sc = pl.pallas_call(
        _dist_body,
        grid=(_ALL_TILES - _TC_TILES,),
        in_specs=[
            pl.BlockSpec((1, N, _C), lambda i: ((i + _TC_TILES) // tpr, 0, 0)),
            pl.BlockSpec((1, _ROWS, _C),
                         lambda i: ((i + _TC_TILES) // tpr,
                                    (i + _TC_TILES) % tpr, 0)),
        ],
        out_specs=pl.BlockSpec((_ROWS, N), lambda i: (i, 0)),
        out_shape=jax.ShapeDtypeStruct((_SC_ROWS, N), jnp.float32),
    )(xt, xt)
    nn_sc = _sc_topk(dist_sc)
    nn_tc = pl.pallas_call(
        _fused_body,
        grid=(_TC_TILES,),
        in_specs=[
            pl.BlockSpec((1, N, _C), lambda i: (i // tpr, 0, 0)),
            pl.BlockSpec((1, _ROWS, _C), lambda i: (i // tpr, i % tpr, 0)),
        ],
        out_specs=pl.BlockSpec((_ROWS, _K), lambda i: (i, 0)),
        out_shape=jax.ShapeDtypeStruct((_SPLIT, _K), jnp.int32),
    )(xt, xt)
    nn = jnp.concatenate([nn_tc, nn_sc], axis=0).reshape(B, N, _K)
    center = jnp.broadcast_to(
        jnp.arange(N, dtype=jnp.int32)[None, :, None], (B, N, _K))
    return jnp.stack([nn, center], axis=0)


# speedup vs baseline: 12.4192x; 1.0014x over previous
import functools

import jax
import jax.numpy as jnp
from jax import lax
from jax.experimental import pallas as pl
from jax.experimental.pallas import tpu as pltpu
from jax.experimental.pallas import tpu_sc as plsc

_K = 16
_B = 4
_C = 32
_N = 4096
_ROWS = 256
_L = 16
_NW = 32

_SPLIT = 7936
_TC_TILES = _SPLIT // _ROWS
_ALL_TILES = (_B * _N) // _ROWS
_SC_ROWS = _B * _N - _SPLIT
_RPW = _SC_ROWS // _NW


def _normalize(xfull_ref, xrows_ref):
    xf = xfull_ref[0]
    xr = xrows_ref[0]
    nf = jnp.sqrt(jnp.sum(xf * xf, axis=-1, keepdims=True))
    xfn = xf / jnp.maximum(nf, 1e-12)
    nr = jnp.sqrt(jnp.sum(xr * xr, axis=-1, keepdims=True))
    xrn = xr / jnp.maximum(nr, 1e-12)
    return xfn, xrn


def _dist_tile(xfn, xrn):
    sqf = jnp.sum(xfn * xfn, axis=-1)
    sqr = jnp.sum(xrn * xrn, axis=-1)
    inner = -2.0 * lax.dot_general(
        xrn, xfn, (((1,), (1,)), ((), ())),
        preferred_element_type=jnp.float32)
    return (sqr[:, None] + inner) + sqf[None, :]


def _dist_body(xfull_ref, xrows_ref, out_ref):
    xfn, xrn = _normalize(xfull_ref, xrows_ref)
    out_ref[...] = _dist_tile(xfn, xrn)


def _fused_body(xfull_ref, xrows_ref, out_ref):
    xfn, xrn = _normalize(xfull_ref, xrows_ref)
    dist = _dist_tile(xfn, xrn)
    R, N = dist.shape
    iota = lax.broadcasted_iota(jnp.int32, (R, N), 1)
    work = dist
    cols = []
    inf = jnp.float32(jnp.inf)
    for t in range(2 * _K - 1):
        m = jnp.min(work, axis=1, keepdims=True)
        idx = jnp.min(jnp.where(work == m, iota, N), axis=1,
                      keepdims=True)
        if t % 2 == 0:
            cols.append(idx)
        if t + 1 < 2 * _K - 1:
            work = jnp.where(iota == idx, inf, work)
    out_ref[...] = jnp.concatenate(cols, axis=1)


def _lexlt(ak, ai, bk, bi):
    return (ak < bk) | ((ak == bk) & (ai < bi))


def _split(ak, ai, bk, bi):
    m = _lexlt(ak, ai, bk, bi)
    lok = jnp.where(m, ak, bk)
    loi = jnp.where(m, ai, bi)
    hik = jnp.where(m, bk, ak)
    hii = jnp.where(m, bi, ai)
    return lok, loi, hik, hii


def _rev(x):
    return lax.rev(x, (0,))


def _merge_chunk(c0k, c0i, c1k, c1i, vk, vi):
    sv, si = plsc.sort_key_val(vk, vi)
    lo1k, lo1i, hi1k, hi1i = _split(c1k, c1i, _rev(sv), _rev(si))
    lo1k, lo1i = plsc.sort_key_val(lo1k, lo1i)
    hi1k, hi1i = plsc.sort_key_val(hi1k, hi1i)
    nc0k, nc0i, hi2k, hi2i = _split(c0k, c0i, _rev(lo1k), _rev(lo1i))
    nc0k, nc0i = plsc.sort_key_val(nc0k, nc0i)
    hi2k, hi2i = plsc.sort_key_val(hi2k, hi2i)
    nc1k, nc1i, _uk, _ui = _split(hi2k, hi2i, _rev(hi1k), _rev(hi1i))
    nc1k, nc1i = plsc.sort_key_val(nc1k, nc1i)
    return nc0k, nc0i, nc1k, nc1i


_GRP = 4
_CAP = 112
_TRIG = 48


def _topk_row(buf, p, r, outb, stage, abk, abi):
    lane = lax.iota(jnp.int32, _L)
    inf = jnp.float32(jnp.inf)

    def consolidate(c0k, c0i, c1k, c1i, ncnt):
        ncnt_s = jnp.broadcast_to(ncnt, (_L,))
        st = (c0k, c0i, c1k, c1i)

        def fold(st, j):
            bk = abk[pl.ds(j * _L, _L)]
            bi = abi[pl.ds(j * _L, _L)]
            valid = (lane + j * _L) < ncnt_s
            bk = jnp.where(valid, bk, inf)
            bi = jnp.where(valid, bi, 0)
            return _merge_chunk(*st, bk, bi)

        for j in range(_CAP // _L):
            st = lax.cond(j * _L < ncnt, lambda s: fold(s, j),
                          lambda s: s, st)
        return st

    def group(g, carry):
        c0k, c0i, c1k, c1i, th, cnt = carry
        vs = [buf[p, pl.ds((g * _GRP + j) * _L, _L)] for j in range(_GRP)]
        gmin = jnp.min(jnp.minimum(jnp.minimum(vs[0], vs[1]),
                                   jnp.minimum(vs[2], vs[3])))

        def hit(op):
            c0k, c0i, c1k, c1i, th, cnt = op
            th_s = jnp.broadcast_to(th, (_L,))
            off = jnp.broadcast_to(cnt, (_L,))
            for j in range(_GRP):
                m = vs[j] <= th_s
                pos = off + plsc.cumsum(m.astype(jnp.int32)) - 1
                plsc.store_scatter(abk, [pos], vs[j], mask=m)
                plsc.store_scatter(abi, [pos],
                                   lane + (g * _GRP + j) * _L, mask=m)
                off = off + plsc.all_reduce_population_count(m)
            ncnt = off[0]

            def consol(op2):
                c0k, c0i, c1k, c1i, _th, _c = op2
                c0k, c0i, c1k, c1i = consolidate(c0k, c0i, c1k, c1i, ncnt)
                return (c0k, c0i, c1k, c1i, jnp.max(c1k),
                        jnp.int32(0))

            return lax.cond(
                ncnt >= _TRIG, consol,
                lambda op2: (op2[0], op2[1], op2[2], op2[3], op2[4], ncnt),
                (c0k, c0i, c1k, c1i, th, cnt))

        return lax.cond(gmin <= th, hit, lambda op: op, carry)

    init = (jnp.full((_L,), inf, jnp.float32), jnp.zeros((_L,), jnp.int32),
            jnp.full((_L,), inf, jnp.float32), jnp.zeros((_L,), jnp.int32),
            inf, jnp.int32(0))
    c0k, c0i, c1k, c1i, th, cnt = lax.fori_loop(
        0, _N // (_GRP * _L), group, init)
    c0k, c0i, c1k, c1i = lax.cond(
        cnt > 0,
        lambda st: consolidate(st[0], st[1], st[2], st[3], cnt),
        lambda st: st, (c0k, c0i, c1k, c1i))
    stage[0] = c0i
    stage[1] = c1i
    ev = plsc.load_gather(stage, [lane // 8, (lane * 2) % _L])
    outb[r] = ev


@functools.cache
def _get_sc_topk():
    mesh = plsc.VectorSubcoreMesh(core_axis_name="c", subcore_axis_name="s",
                                  num_cores=2)

    @functools.partial(
        pl.kernel,
        mesh=mesh,
        compiler_params=pltpu.CompilerParams(needs_layout_passes=False),
        out_type=jax.ShapeDtypeStruct((_SC_ROWS, _L), jnp.int32),
        scratch_types=[
            pltpu.VMEM((2, _N), jnp.float32),
            pltpu.VMEM((_RPW, _L), jnp.int32),
            pltpu.VMEM((2, _L), jnp.int32),
            pltpu.VMEM((_CAP,), jnp.float32),
            pltpu.VMEM((_CAP,), jnp.int32),
            pltpu.SemaphoreType.DMA,
            pltpu.SemaphoreType.DMA,
        ],
    )
    def _sc_topk(dist_hbm, out_hbm, buf, outb, stage, abk, abi, sem0, sem1):
        cid = lax.axis_index("c")
        sid = lax.axis_index("s")
        wid = sid * 2 + cid
        base = wid * _RPW
        pltpu.async_copy(dist_hbm.at[base], buf.at[0], sem0)

        def row_loop(g, carry):
            for p in range(2):
                r = g * 2 + p
                sem_cur = sem0 if p == 0 else sem1
                sem_nxt = sem1 if p == 0 else sem0

                @pl.when(r + 1 < _RPW)
                def _():
                    pltpu.async_copy(dist_hbm.at[base + r + 1], buf.at[1 - p],
                                     sem_nxt)

                pltpu.make_async_copy(dist_hbm.at[base + r], buf.at[p],
                                      sem_cur).wait()
                _topk_row(buf, p, r, outb, stage, abk, abi)
            return carry

        lax.fori_loop(0, _RPW // 2, row_loop, 0)
        pltpu.sync_copy(outb, out_hbm.at[pl.ds(base, _RPW)])

    return _sc_topk


def kernel(x):
    B, C, N, _ = x.shape
    xt = jnp.squeeze(jnp.transpose(x, (0, 2, 1, 3)), -1)
    tpr = N // _ROWS
    dist_sc = pl.pallas_call(
        _dist_body,
        grid=(_ALL_TILES - _TC_TILES,),
        in_specs=[
            pl.BlockSpec((1, N, _C), lambda i: ((i + _TC_TILES) // tpr, 0, 0)),
            pl.BlockSpec((1, _ROWS, _C),
                         lambda i: ((i + _TC_TILES) // tpr,
                                    (i + _TC_TILES) % tpr, 0)),
        ],
        out_specs=pl.BlockSpec((_ROWS, N), lambda i: (i, 0)),
        out_shape=jax.ShapeDtypeStruct((_SC_ROWS, N), jnp.float32),
    )(xt, xt)
    nn_sc = _get_sc_topk()(dist_sc)
    nn_tc = pl.pallas_call(
        _fused_body,
        grid=(_TC_TILES,),
        in_specs=[
            pl.BlockSpec((1, N, _C), lambda i: (i // tpr, 0, 0)),
            pl.BlockSpec((1, _ROWS, _C), lambda i: (i // tpr, i % tpr, 0)),
        ],
        out_specs=pl.BlockSpec((_ROWS, _K), lambda i: (i, 0)),
        out_shape=jax.ShapeDtypeStruct((_SPLIT, _K), jnp.int32),
    )(xt, xt)
    nn = jnp.concatenate([nn_tc, nn_sc], axis=0).reshape(B, N, _K)
    center = jnp.broadcast_to(
        jnp.arange(N, dtype=jnp.int32)[None, :, None], (B, N, _K))
    return jnp.stack([nn, center], axis=0)


# speedup vs baseline: 12.4194x; 1.0000x over previous
import functools

import jax
import jax.numpy as jnp
from jax import lax
from jax.experimental import pallas as pl
from jax.experimental.pallas import tpu as pltpu
from jax.experimental.pallas import tpu_sc as plsc

_K = 16
_B = 4
_C = 32
_N = 4096
_ROWS = 256
_L = 16
_NW = 32

_SPLIT = 7936
_TC_TILES = _SPLIT // _ROWS
_ALL_TILES = (_B * _N) // _ROWS
_SC_ROWS = _B * _N - _SPLIT
_RPW = _SC_ROWS // _NW


def _normalize(xfull_ref, xrows_ref):
    xf = xfull_ref[0]
    xr = xrows_ref[0]
    nf = jnp.sqrt(jnp.sum(xf * xf, axis=-1, keepdims=True))
    xfn = xf / jnp.maximum(nf, 1e-12)
    nr = jnp.sqrt(jnp.sum(xr * xr, axis=-1, keepdims=True))
    xrn = xr / jnp.maximum(nr, 1e-12)
    return xfn, xrn


def _dist_tile(xfn, xrn):
    sqf = jnp.sum(xfn * xfn, axis=-1)
    sqr = jnp.sum(xrn * xrn, axis=-1)
    inner = -2.0 * lax.dot_general(
        xrn, xfn, (((1,), (1,)), ((), ())),
        preferred_element_type=jnp.float32)
    return (sqr[:, None] + inner) + sqf[None, :]


def _dist_body(xfull_ref, xrows_ref, out_ref):
    xfn, xrn = _normalize(xfull_ref, xrows_ref)
    out_ref[...] = _dist_tile(xfn, xrn)


def _fused_body(xfull_ref, xrows_ref, out_ref):
    xfn, xrn = _normalize(xfull_ref, xrows_ref)
    dist = _dist_tile(xfn, xrn)
    R, N = dist.shape
    iota = lax.broadcasted_iota(jnp.int32, (R, N), 1)
    work = dist
    cols = []
    inf = jnp.float32(jnp.inf)
    for t in range(2 * _K - 1):
        m = jnp.min(work, axis=1, keepdims=True)
        idx = jnp.min(jnp.where(work == m, iota, N), axis=1,
                      keepdims=True)
        if t % 2 == 0:
            cols.append(idx)
        if t + 1 < 2 * _K - 1:
            work = jnp.where(iota == idx, inf, work)
    out_ref[...] = jnp.concatenate(cols, axis=1)


def _lexlt(ak, ai, bk, bi):
    return (ak < bk) | ((ak == bk) & (ai < bi))


def _split(ak, ai, bk, bi):
    m = _lexlt(ak, ai, bk, bi)
    lok = jnp.where(m, ak, bk)
    loi = jnp.where(m, ai, bi)
    hik = jnp.where(m, bk, ak)
    hii = jnp.where(m, bi, ai)
    return lok, loi, hik, hii


def _rev(x):
    return lax.rev(x, (0,))


def _merge_chunk(c0k, c0i, c1k, c1i, vk, vi):
    sv, si = plsc.sort_key_val(vk, vi)
    lo1k, lo1i, hi1k, hi1i = _split(c1k, c1i, _rev(sv), _rev(si))
    lo1k, lo1i = plsc.sort_key_val(lo1k, lo1i)
    hi1k, hi1i = plsc.sort_key_val(hi1k, hi1i)
    nc0k, nc0i, hi2k, hi2i = _split(c0k, c0i, _rev(lo1k), _rev(lo1i))
    nc0k, nc0i = plsc.sort_key_val(nc0k, nc0i)
    hi2k, hi2i = plsc.sort_key_val(hi2k, hi2i)
    nc1k, nc1i, _uk, _ui = _split(hi2k, hi2i, _rev(hi1k), _rev(hi1i))
    nc1k, nc1i = plsc.sort_key_val(nc1k, nc1i)
    return nc0k, nc0i, nc1k, nc1i


_GRP = 4
_CAP = 112
_TRIG = 48


def _topk_row(buf, p, r, outb, stage, abk, abi):
    lane = lax.iota(jnp.int32, _L)
    inf = jnp.float32(jnp.inf)

    def consolidate(c0k, c0i, c1k, c1i, ncnt):
        ncnt_s = jnp.broadcast_to(ncnt, (_L,))
        st = (c0k, c0i, c1k, c1i)

        def fold(st, j):
            bk = abk[pl.ds(j * _L, _L)]
            bi = abi[pl.ds(j * _L, _L)]
            valid = (lane + j * _L) < ncnt_s
            bk = jnp.where(valid, bk, inf)
            bi = jnp.where(valid, bi, 0)
            return _merge_chunk(*st, bk, bi)

        for j in range(_CAP // _L):
            st = lax.cond(j * _L < ncnt, lambda s: fold(s, j),
                          lambda s: s, st)
        return st

    def group(g, carry):
        c0k, c0i, c1k, c1i, th, cnt = carry
        vs = [buf[p, pl.ds((g * _GRP + j) * _L, _L)] for j in range(_GRP)]
        th_s = jnp.broadcast_to(th, (_L,))
        ms = [vs[j] <= th_s for j in range(_GRP)]
        anym = (ms[0] | ms[1]) | (ms[2] | ms[3])
        hitcnt = plsc.all_reduce_population_count(anym)[0]

        def hit(op):
            c0k, c0i, c1k, c1i, th, cnt = op
            off = jnp.broadcast_to(cnt, (_L,))
            sums = [plsc.cumsum(ms[j].astype(jnp.int32)) for j in range(_GRP)]
            pcs = [plsc.all_reduce_population_count(ms[j])
                   for j in range(_GRP)]
            for j in range(_GRP):
                pos = off + sums[j] - 1
                plsc.store_scatter(abk, [pos], vs[j], mask=ms[j])
                plsc.store_scatter(abi, [pos],
                                   lane + (g * _GRP + j) * _L, mask=ms[j])
                off = off + pcs[j]
            ncnt = off[0]

            def consol(op2):
                c0k, c0i, c1k, c1i, _th, _c = op2
                c0k, c0i, c1k, c1i = consolidate(c0k, c0i, c1k, c1i, ncnt)
                return (c0k, c0i, c1k, c1i, jnp.max(c1k),
                        jnp.int32(0))

            return lax.cond(
                ncnt >= _TRIG, consol,
                lambda op2: (op2[0], op2[1], op2[2], op2[3], op2[4], ncnt),
                (c0k, c0i, c1k, c1i, th, cnt))

        return lax.cond(hitcnt > 0, hit, lambda op: op, carry)

    init = (jnp.full((_L,), inf, jnp.float32), jnp.zeros((_L,), jnp.int32),
            jnp.full((_L,), inf, jnp.float32), jnp.zeros((_L,), jnp.int32),
            inf, jnp.int32(0))
    c0k, c0i, c1k, c1i, th, cnt = lax.fori_loop(
        0, _N // (_GRP * _L), group, init)
    c0k, c0i, c1k, c1i = lax.cond(
        cnt > 0,
        lambda st: consolidate(st[0], st[1], st[2], st[3], cnt),
        lambda st: st, (c0k, c0i, c1k, c1i))
    stage[0] = c0i
    stage[1] = c1i
    ev = plsc.load_gather(stage, [lane // 8, (lane * 2) % _L])
    outb[r] = ev


@functools.cache
def _get_sc_topk():
    mesh = plsc.VectorSubcoreMesh(core_axis_name="c", subcore_axis_name="s",
                                  num_cores=2)

    @functools.partial(
        pl.kernel,
        mesh=mesh,
        compiler_params=pltpu.CompilerParams(needs_layout_passes=False),
        out_type=jax.ShapeDtypeStruct((_SC_ROWS, _L), jnp.int32),
        scratch_types=[
            pltpu.VMEM((2, _N), jnp.float32),
            pltpu.VMEM((_RPW, _L), jnp.int32),
            pltpu.VMEM((2, _L), jnp.int32),
            pltpu.VMEM((_CAP,), jnp.float32),
            pltpu.VMEM((_CAP,), jnp.int32),
            pltpu.SemaphoreType.DMA,
            pltpu.SemaphoreType.DMA,
        ],
    )
    def _sc_topk(dist_hbm, out_hbm, buf, outb, stage, abk, abi, sem0, sem1):
        cid = lax.axis_index("c")
        sid = lax.axis_index("s")
        wid = sid * 2 + cid
        base = wid * _RPW
        pltpu.async_copy(dist_hbm.at[base], buf.at[0], sem0)

        def row_loop(g, carry):
            for p in range(2):
                r = g * 2 + p
                sem_cur = sem0 if p == 0 else sem1
                sem_nxt = sem1 if p == 0 else sem0

                @pl.when(r + 1 < _RPW)
                def _():
                    pltpu.async_copy(dist_hbm.at[base + r + 1], buf.at[1 - p],
                                     sem_nxt)

                pltpu.make_async_copy(dist_hbm.at[base + r], buf.at[p],
                                      sem_cur).wait()
                _topk_row(buf, p, r, outb, stage, abk, abi)
            return carry

        lax.fori_loop(0, _RPW // 2, row_loop, 0)
        pltpu.sync_copy(outb, out_hbm.at[pl.ds(base, _RPW)])

    return _sc_topk


def kernel(x):
    B, C, N, _ = x.shape
    xt = jnp.squeeze(jnp.transpose(x, (0, 2, 1, 3)), -1)
    tpr = N // _ROWS
    dist_sc = pl.pallas_call(
        _dist_body,
        grid=(_ALL_TILES - _TC_TILES,),
        in_specs=[
            pl.BlockSpec((1, N, _C), lambda i: ((i + _TC_TILES) // tpr, 0, 0)),
            pl.BlockSpec((1, _ROWS, _C),
                         lambda i: ((i + _TC_TILES) // tpr,
                                    (i + _TC_TILES) % tpr, 0)),
        ],
        out_specs=pl.BlockSpec((_ROWS, N), lambda i: (i, 0)),
        out_shape=jax.ShapeDtypeStruct((_SC_ROWS, N), jnp.float32),
    )(xt, xt)
    nn_sc = _get_sc_topk()(dist_sc)
    nn_tc = pl.pallas_call(
        _fused_body,
        grid=(_TC_TILES,),
        in_specs=[
            pl.BlockSpec((1, N, _C), lambda i: (i // tpr, 0, 0)),
            pl.BlockSpec((1, _ROWS, _C), lambda i: (i // tpr, i % tpr, 0)),
        ],
        out_specs=pl.BlockSpec((_ROWS, _K), lambda i: (i, 0)),
        out_shape=jax.ShapeDtypeStruct((_SPLIT, _K), jnp.int32),
    )(xt, xt)
    nn = jnp.concatenate([nn_tc, nn_sc], axis=0).reshape(B, N, _K)
    center = jnp.broadcast_to(
        jnp.arange(N, dtype=jnp.int32)[None, :, None], (B, N, _K))
    return jnp.stack([nn, center], axis=0)


# speedup vs baseline: 12.4404x; 1.0017x over previous
import functools

import jax
import jax.numpy as jnp
from jax import lax
from jax.experimental import pallas as pl
from jax.experimental.pallas import tpu as pltpu
from jax.experimental.pallas import tpu_sc as plsc

_K = 16
_B = 4
_C = 32
_N = 4096
_ROWS = 256
_L = 16
_NW = 32

_SPLIT = 7168
_TC_TILES = _SPLIT // _ROWS
_ALL_TILES = (_B * _N) // _ROWS
_SC_ROWS = _B * _N - _SPLIT
_RPW = _SC_ROWS // _NW


def _normalize(xfull_ref, xrows_ref):
    xf = xfull_ref[0]
    xr = xrows_ref[0]
    nf = jnp.sqrt(jnp.sum(xf * xf, axis=-1, keepdims=True))
    xfn = xf / jnp.maximum(nf, 1e-12)
    nr = jnp.sqrt(jnp.sum(xr * xr, axis=-1, keepdims=True))
    xrn = xr / jnp.maximum(nr, 1e-12)
    return xfn, xrn


def _dist_tile(xfn, xrn):
    sqf = jnp.sum(xfn * xfn, axis=-1)
    sqr = jnp.sum(xrn * xrn, axis=-1)
    inner = -2.0 * lax.dot_general(
        xrn, xfn, (((1,), (1,)), ((), ())),
        preferred_element_type=jnp.float32)
    return (sqr[:, None] + inner) + sqf[None, :]


def _dist_body(xfull_ref, xrows_ref, out_ref):
    xfn, xrn = _normalize(xfull_ref, xrows_ref)
    out_ref[...] = _dist_tile(xfn, xrn)


def _fused_body(xfull_ref, xrows_ref, out_ref):
    xfn, xrn = _normalize(xfull_ref, xrows_ref)
    dist = _dist_tile(xfn, xrn)
    R, N = dist.shape
    iota = lax.broadcasted_iota(jnp.int32, (R, N), 1)
    work = dist
    cols = []
    inf = jnp.float32(jnp.inf)
    for t in range(2 * _K - 1):
        m = jnp.min(work, axis=1, keepdims=True)
        idx = jnp.min(jnp.where(work == m, iota, N), axis=1,
                      keepdims=True)
        if t % 2 == 0:
            cols.append(idx)
        if t + 1 < 2 * _K - 1:
            work = jnp.where(iota == idx, inf, work)
    out_ref[...] = jnp.concatenate(cols, axis=1)


def _lexlt(ak, ai, bk, bi):
    return (ak < bk) | ((ak == bk) & (ai < bi))


def _split(ak, ai, bk, bi):
    m = _lexlt(ak, ai, bk, bi)
    lok = jnp.where(m, ak, bk)
    loi = jnp.where(m, ai, bi)
    hik = jnp.where(m, bk, ak)
    hii = jnp.where(m, bi, ai)
    return lok, loi, hik, hii


def _rev(x):
    return lax.rev(x, (0,))


def _merge_chunk(c0k, c0i, c1k, c1i, vk, vi):
    sv, si = plsc.sort_key_val(vk, vi)
    lo1k, lo1i, hi1k, hi1i = _split(c1k, c1i, _rev(sv), _rev(si))
    lo1k, lo1i = plsc.sort_key_val(lo1k, lo1i)
    hi1k, hi1i = plsc.sort_key_val(hi1k, hi1i)
    nc0k, nc0i, hi2k, hi2i = _split(c0k, c0i, _rev(lo1k), _rev(lo1i))
    nc0k, nc0i = plsc.sort_key_val(nc0k, nc0i)
    hi2k, hi2i = plsc.sort_key_val(hi2k, hi2i)
    nc1k, nc1i, _uk, _ui = _split(hi2k, hi2i, _rev(hi1k), _rev(hi1i))
    nc1k, nc1i = plsc.sort_key_val(nc1k, nc1i)
    return nc0k, nc0i, nc1k, nc1i


_GRP = 4
_CAP = 112
_TRIG = 48


def _topk_row(buf, p, r, outb, stage, abk, abi):
    lane = lax.iota(jnp.int32, _L)
    inf = jnp.float32(jnp.inf)

    def consolidate(c0k, c0i, c1k, c1i, ncnt):
        ncnt_s = jnp.broadcast_to(ncnt, (_L,))
        st = (c0k, c0i, c1k, c1i)

        def fold(st, j):
            bk = abk[pl.ds(j * _L, _L)]
            bi = abi[pl.ds(j * _L, _L)]
            valid = (lane + j * _L) < ncnt_s
            bk = jnp.where(valid, bk, inf)
            bi = jnp.where(valid, bi, 0)
            return _merge_chunk(*st, bk, bi)

        for j in range(_CAP // _L):
            st = lax.cond(j * _L < ncnt, lambda s: fold(s, j),
                          lambda s: s, st)
        return st

    def group(g, carry):
        c0k, c0i, c1k, c1i, th, cnt = carry
        vs = [buf[p, pl.ds((g * _GRP + j) * _L, _L)] for j in range(_GRP)]
        th_s = jnp.broadcast_to(th, (_L,))
        ms = [vs[j] <= th_s for j in range(_GRP)]
        anym = (ms[0] | ms[1]) | (ms[2] | ms[3])
        hitcnt = plsc.all_reduce_population_count(anym)[0]

        def hit(op):
            c0k, c0i, c1k, c1i, th, cnt = op
            off = jnp.broadcast_to(cnt, (_L,))
            sums = [plsc.cumsum(ms[j].astype(jnp.int32)) for j in range(_GRP)]
            pcs = [plsc.all_reduce_population_count(ms[j])
                   for j in range(_GRP)]
            for j in range(_GRP):
                pos = off + sums[j] - 1
                plsc.store_scatter(abk, [pos], vs[j], mask=ms[j])
                plsc.store_scatter(abi, [pos],
                                   lane + (g * _GRP + j) * _L, mask=ms[j])
                off = off + pcs[j]
            ncnt = off[0]

            def consol(op2):
                c0k, c0i, c1k, c1i, _th, _c = op2
                c0k, c0i, c1k, c1i = consolidate(c0k, c0i, c1k, c1i, ncnt)
                return (c0k, c0i, c1k, c1i, jnp.max(c1k),
                        jnp.int32(0))

            return lax.cond(
                ncnt >= _TRIG, consol,
                lambda op2: (op2[0], op2[1], op2[2], op2[3], op2[4], ncnt),
                (c0k, c0i, c1k, c1i, th, cnt))

        return lax.cond(hitcnt > 0, hit, lambda op: op, carry)

    init = (jnp.full((_L,), inf, jnp.float32), jnp.zeros((_L,), jnp.int32),
            jnp.full((_L,), inf, jnp.float32), jnp.zeros((_L,), jnp.int32),
            inf, jnp.int32(0))
    c0k, c0i, c1k, c1i, th, cnt = lax.fori_loop(
        0, _N // (_GRP * _L), group, init)
    c0k, c0i, c1k, c1i = lax.cond(
        cnt > 0,
        lambda st: consolidate(st[0], st[1], st[2], st[3], cnt),
        lambda st: st, (c0k, c0i, c1k, c1i))
    stage[0] = c0i
    stage[1] = c1i
    ev = plsc.load_gather(stage, [lane // 8, (lane * 2) % _L])
    outb[r] = ev


@functools.cache
def _get_sc_topk():
    mesh = plsc.VectorSubcoreMesh(core_axis_name="c", subcore_axis_name="s",
                                  num_cores=2)

    @functools.partial(
        pl.kernel,
        mesh=mesh,
        compiler_params=pltpu.CompilerParams(needs_layout_passes=False),
        out_type=jax.ShapeDtypeStruct((_SC_ROWS, _L), jnp.int32),
        scratch_types=[
            pltpu.VMEM((2, _N), jnp.float32),
            pltpu.VMEM((_RPW, _L), jnp.int32),
            pltpu.VMEM((2, _L), jnp.int32),
            pltpu.VMEM((_CAP,), jnp.float32),
            pltpu.VMEM((_CAP,), jnp.int32),
            pltpu.SemaphoreType.DMA,
            pltpu.SemaphoreType.DMA,
        ],
    )
    def _sc_topk(dist_hbm, out_hbm, buf, outb, stage, abk, abi, sem0, sem1):
        cid = lax.axis_index("c")
        sid = lax.axis_index("s")
        wid = sid * 2 + cid
        base = wid * _RPW
        pltpu.async_copy(dist_hbm.at[base], buf.at[0], sem0)

        def row_loop(g, carry):
            for p in range(2):
                r = g * 2 + p
                sem_cur = sem0 if p == 0 else sem1
                sem_nxt = sem1 if p == 0 else sem0

                @pl.when(r + 1 < _RPW)
                def _():
                    pltpu.async_copy(dist_hbm.at[base + r + 1], buf.at[1 - p],
                                     sem_nxt)

                pltpu.make_async_copy(dist_hbm.at[base + r], buf.at[p],
                                      sem_cur).wait()
                _topk_row(buf, p, r, outb, stage, abk, abi)
            return carry

        lax.fori_loop(0, _RPW // 2, row_loop, 0)
        pltpu.sync_copy(outb, out_hbm.at[pl.ds(base, _RPW)])

    return _sc_topk


def kernel(x):
    B, C, N, _ = x.shape
    xt = jnp.squeeze(jnp.transpose(x, (0, 2, 1, 3)), -1)
    tpr = N // _ROWS
    dist_sc = pl.pallas_call(
        _dist_body,
        grid=(_ALL_TILES - _TC_TILES,),
        in_specs=[
            pl.BlockSpec((1, N, _C), lambda i: ((i + _TC_TILES) // tpr, 0, 0)),
            pl.BlockSpec((1, _ROWS, _C),
                         lambda i: ((i + _TC_TILES) // tpr,
                                    (i + _TC_TILES) % tpr, 0)),
        ],
        out_specs=pl.BlockSpec((_ROWS, N), lambda i: (i, 0)),
        out_shape=jax.ShapeDtypeStruct((_SC_ROWS, N), jnp.float32),
    )(xt, xt)
    nn_sc = _get_sc_topk()(dist_sc)
    nn_tc = pl.pallas_call(
        _fused_body,
        grid=(_TC_TILES,),
        in_specs=[
            pl.BlockSpec((1, N, _C), lambda i: (i // tpr, 0, 0)),
            pl.BlockSpec((1, _ROWS, _C), lambda i: (i // tpr, i % tpr, 0)),
        ],
        out_specs=pl.BlockSpec((_ROWS, _K), lambda i: (i, 0)),
        out_shape=jax.ShapeDtypeStruct((_SPLIT, _K), jnp.int32),
    )(xt, xt)
    nn = jnp.concatenate([nn_tc, nn_sc], axis=0).reshape(B, N, _K)
    center = jnp.broadcast_to(
        jnp.arange(N, dtype=jnp.int32)[None, :, None], (B, N, _K))
    return jnp.stack([nn, center], axis=0)
